# baseline (speedup 1.0000x reference)
import functools

import jax
import jax.numpy as jnp
from jax import lax
from jax.experimental import pallas as pl
from jax.experimental.pallas import tpu as pltpu
from jax.experimental.pallas import tpu_sc as plsc

NUM_ENTITIES = 1000000
ENTITY_DIM = 32
BATCH = 16384

_INFO = plsc.get_sparse_core_info()
NC = _INFO.num_cores
NS = _INFO.num_subcores
NW = NC * NS
B_PER_W = BATCH // NW
IDX_CHUNK = 128
N_CHUNKS = B_PER_W // IDX_CHUNK


@functools.partial(
    pl.kernel,
    mesh=plsc.VectorSubcoreMesh(core_axis_name="c", subcore_axis_name="s"),
    compiler_params=pltpu.CompilerParams(use_tc_tiling_on_sc=False),
    out_type=jax.ShapeDtypeStruct((BATCH, ENTITY_DIM), jnp.float32),
    scratch_types=[
        pltpu.VMEM((B_PER_W,), jnp.int32),
        pltpu.VMEM((B_PER_W, ENTITY_DIM), jnp.float32),
        pltpu.SemaphoreType.DMA,
    ],
)
def _gather_sc(idx_hbm, table_hbm, out_hbm, idx_v, rows_v, sem):
    wid = lax.axis_index("s") * NC + lax.axis_index("c")
    base = wid * B_PER_W
    pltpu.sync_copy(idx_hbm.at[pl.ds(base, B_PER_W)], idx_v)
    copies = [
        pltpu.async_copy(
            table_hbm.at[idx_v.at[pl.ds(j * IDX_CHUNK, IDX_CHUNK)]],
            rows_v.at[pl.ds(j * IDX_CHUNK, IDX_CHUNK)],
            sem,
        )
        for j in range(N_CHUNKS)
    ]
    for c in copies:
        c.wait()
    pltpu.sync_copy(rows_v, out_hbm.at[pl.ds(base, B_PER_W)])


def kernel(indexes, entity_table):
    return _gather_sc(indexes.astype(jnp.int32), entity_table)
